# baseline (speedup 1.0000x reference)
import functools

import jax
import jax.numpy as jnp
from jax import lax
from jax.experimental import pallas as pl
from jax.experimental.pallas import tpu as pltpu
from jax.experimental.pallas import tpu_sc as plsc

GROUPS = 4
V = 1024
B = 1024
D = 256
ROW = D * D
KS = 16
CH = ROW // KS
NIDX = 8
NC = 2
NS = 16
L = 16
NW = NC * NS
BPW = B // NW
NDMA = BPW * (KS // NIDX)

_mesh = plsc.VectorSubcoreMesh(core_axis_name="c", subcore_axis_name="s")


@functools.partial(
    pl.kernel,
    mesh=_mesh,
    out_type=(
        jax.ShapeDtypeStruct((V * KS, CH), jnp.float32),
        jax.ShapeDtypeStruct((B, D), jnp.float32),
    ),
    scratch_types=[
        pltpu.VMEM((L,), jnp.int32),
        pltpu.VMEM((L,), jnp.int32),
        pltpu.VMEM((BPW,), jnp.int32),
        pltpu.VMEM((BPW,), jnp.int32),
        pltpu.VMEM((BPW * KS,), jnp.int32),
        pltpu.VMEM((BPW * KS,), jnp.int32),
        pltpu.VMEM((BPW, D), jnp.float32),
        pltpu.VMEM((NIDX, CH), jnp.float32),
        pltpu.VMEM((NIDX, CH), jnp.float32),
        pltpu.SemaphoreType.DMA,
        pltpu.SemaphoreType.DMA,
        pltpu.SemaphoreType.DMA,
        pltpu.SemaphoreType.DMA,
        pltpu.SemaphoreType.DMA,
    ],
)
def _sc_lookup(ld, selflat, wtab, btab, outw, outb,
               z_v, g_v, sidx_v, widx_v, rep_v, idx_v, bbuf, wbuf0, wbuf1,
               usem, gs0, gs1, ps0, ps1):
    wid = lax.axis_index("s") * NC + lax.axis_index("c")
    base = pl.multiple_of(wid * BPW, BPW)
    iota = lax.iota(jnp.int32, L)

    z_v[...] = iota * 0
    pltpu.async_copy(ld.at[z_v], g_v, usem).wait()
    g = g_v[...]

    for h in range(BPW // L):
        sidx_v[pl.ds(h * L, L)] = (base + h * L + iota) * GROUPS + g
    pltpu.async_copy(selflat.at[sidx_v], widx_v, usem).wait()

    pltpu.async_copy(btab.at[widx_v], bbuf, usem).wait()
    pltpu.sync_copy(bbuf, outb.at[pl.ds(base, BPW)])

    for i in range(BPW):
        rep_v[pl.ds(i * KS, L)] = iota * 0 + ((base + i) * GROUPS) + g
    pltpu.async_copy(selflat.at[rep_v], idx_v, usem).wait()
    for i in range(BPW):
        idx_v[pl.ds(i * KS, L)] = idx_v[pl.ds(i * KS, L)] * KS + iota

    bufs = (wbuf0, wbuf1)
    gsems = (gs0, gs1)
    psems = (ps0, ps1)

    def g_desc(d, b):
        off = pl.multiple_of(d * NIDX, NIDX)
        return pltpu.make_async_copy(
            wtab.at[idx_v.at[pl.ds(off, NIDX)]], bufs[b], gsems[b])

    def p_desc(d, b):
        off = pl.multiple_of(wid * (BPW * KS) + d * NIDX, NIDX)
        return pltpu.make_async_copy(
            bufs[b], outw.at[pl.ds(off, NIDX)], psems[b])

    g_desc(0, 0).start()
    g_desc(1, 1).start()

    def step(i, carry):
        for b in range(2):
            d = i * 2 + b
            g_desc(d, b).wait()
            p_desc(d, b).start()
        for b in range(2):
            d = i * 2 + b
            dn = d + 2

            @pl.when(dn < NDMA)
            def _():
                p_desc(d, b).wait()
                g_desc(dn, b).start()
        return carry

    lax.fori_loop(0, NDMA // 2, step, 0)
    p_desc(NDMA - 2, 0).wait()
    p_desc(NDMA - 1, 1).wait()


def kernel(layer_index, selector, W0, b0, layers_distribution):
    ld = lax.dynamic_slice_in_dim(layers_distribution, layer_index, 1)
    wtab = W0.reshape(V * KS, CH)
    selflat = selector.reshape(B * GROUPS)
    outw, outb = _sc_lookup(ld, selflat, wtab, b0)
    return (outw.reshape(B, D, D), outb)


# speedup vs baseline: 3.8328x; 3.8328x over previous
import functools

import jax
import jax.numpy as jnp
from jax import lax
from jax.experimental import pallas as pl
from jax.experimental.pallas import tpu as pltpu
from jax.experimental.pallas import tpu_sc as plsc

GROUPS = 4
V = 1024
B = 1024
D = 256
EG = 8
SR = 16
NSL = D // SR
NC = 2
NS = 16
L = 16
NW = NC * NS
BPW = B // NW
NDMA = (BPW // EG) * NSL

_mesh = plsc.VectorSubcoreMesh(core_axis_name="c", subcore_axis_name="s")


@functools.partial(
    pl.kernel,
    mesh=_mesh,
    out_type=(
        jax.ShapeDtypeStruct((B, D, D), jnp.float32),
        jax.ShapeDtypeStruct((B, D), jnp.float32),
    ),
    scratch_types=[
        pltpu.VMEM((L,), jnp.int32),
        pltpu.VMEM((L,), jnp.int32),
        pltpu.VMEM((BPW,), jnp.int32),
        pltpu.VMEM((BPW,), jnp.int32),
        pltpu.VMEM((BPW, D), jnp.float32),
        pltpu.VMEM((EG, SR, D), jnp.float32),
        pltpu.VMEM((EG, SR, D), jnp.float32),
        pltpu.SemaphoreType.DMA,
        pltpu.SemaphoreType.DMA,
        pltpu.SemaphoreType.DMA,
        pltpu.SemaphoreType.DMA,
        pltpu.SemaphoreType.DMA,
    ],
)
def _sc_lookup(ld, selflat, wtab, btab, outw, outb,
               z_v, g_v, sidx_v, widx_v, bbuf, wbuf0, wbuf1,
               usem, gs0, gs1, ps0, ps1):
    wid = lax.axis_index("s") * NC + lax.axis_index("c")
    base = pl.multiple_of(wid * BPW, BPW)
    iota = lax.iota(jnp.int32, L)

    z_v[...] = iota * 0
    pltpu.async_copy(ld.at[z_v], g_v, usem).wait()
    g = g_v[...]

    for h in range(BPW // L):
        sidx_v[pl.ds(h * L, L)] = (base + h * L + iota) * GROUPS + g
    pltpu.async_copy(selflat.at[sidx_v], widx_v, usem).wait()

    pltpu.async_copy(btab.at[widx_v], bbuf, usem).wait()
    pltpu.sync_copy(bbuf, outb.at[pl.ds(base, BPW)])

    bufs = (wbuf0, wbuf1)
    gsems = (gs0, gs1)
    psems = (ps0, ps1)

    def g_desc(d, b):
        o = pl.multiple_of((d // NSL) * EG, EG)
        c = pl.multiple_of((d % NSL) * SR, SR)
        return pltpu.make_async_copy(
            wtab.at[widx_v.at[pl.ds(o, EG)], pl.ds(c, SR)], bufs[b], gsems[b])

    def p_desc(d, b):
        o = pl.multiple_of((d // NSL) * EG, EG)
        c = pl.multiple_of((d % NSL) * SR, SR)
        return pltpu.make_async_copy(
            bufs[b], outw.at[pl.ds(base + o, EG), pl.ds(c, SR)], psems[b])

    g_desc(0, 0).start()
    g_desc(1, 1).start()

    def step(i, carry):
        for b in range(2):
            d = i * 2 + b
            g_desc(d, b).wait()
            p_desc(d, b).start()
        for b in range(2):
            d = i * 2 + b
            dn = d + 2

            @pl.when(dn < NDMA)
            def _():
                p_desc(d, b).wait()
                g_desc(dn, b).start()
        return carry

    lax.fori_loop(0, NDMA // 2, step, 0)
    p_desc(NDMA - 2, 0).wait()
    p_desc(NDMA - 1, 1).wait()


def kernel(layer_index, selector, W0, b0, layers_distribution):
    ld = lax.dynamic_slice_in_dim(layers_distribution, layer_index, 1)
    selflat = selector.reshape(B * GROUPS)
    outw, outb = _sc_lookup(ld, selflat, W0, b0)
    return (outw, outb)


# speedup vs baseline: 3.8349x; 1.0005x over previous
import functools

import jax
import jax.numpy as jnp
from jax import lax
from jax.experimental import pallas as pl
from jax.experimental.pallas import tpu as pltpu
from jax.experimental.pallas import tpu_sc as plsc

GROUPS = 4
V = 1024
B = 1024
D = 256
EG = 8
SR = 16
NSL = D // SR
NC = 2
NS = 16
L = 16
NW = NC * NS
BPW = B // NW
NDMA = (BPW // EG) * NSL

_mesh = plsc.VectorSubcoreMesh(core_axis_name="c", subcore_axis_name="s")


@functools.partial(
    pl.kernel,
    mesh=_mesh,
    out_type=(
        jax.ShapeDtypeStruct((B, D, D), jnp.float32),
        jax.ShapeDtypeStruct((B, D), jnp.float32),
    ),
    scratch_types=[
        pltpu.VMEM((L,), jnp.int32),
        pltpu.VMEM((L,), jnp.int32),
        pltpu.VMEM((BPW,), jnp.int32),
        pltpu.VMEM((BPW,), jnp.int32),
        pltpu.VMEM((BPW, D), jnp.float32),
        pltpu.VMEM((EG, SR, D), jnp.float32),
        pltpu.VMEM((EG, SR, D), jnp.float32),
        pltpu.VMEM((EG, SR, D), jnp.float32),
        pltpu.SemaphoreType.DMA,
        pltpu.SemaphoreType.DMA,
        pltpu.SemaphoreType.DMA,
        pltpu.SemaphoreType.DMA,
        pltpu.SemaphoreType.DMA,
        pltpu.SemaphoreType.DMA,
        pltpu.SemaphoreType.DMA,
    ],
)
def _sc_lookup(ld, selflat, wtab, btab, outw, outb,
               z_v, g_v, sidx_v, widx_v, bbuf, wbuf0, wbuf1, wbuf2,
               usem, gs0, gs1, gs2, ps0, ps1, ps2):
    wid = lax.axis_index("s") * NC + lax.axis_index("c")
    base = pl.multiple_of(wid * BPW, BPW)
    iota = lax.iota(jnp.int32, L)

    z_v[...] = iota * 0
    pltpu.async_copy(ld.at[z_v], g_v, usem).wait()
    g = g_v[...]

    for h in range(BPW // L):
        sidx_v[pl.ds(h * L, L)] = (base + h * L + iota) * GROUPS + g
    pltpu.async_copy(selflat.at[sidx_v], widx_v, usem).wait()

    pltpu.async_copy(btab.at[widx_v], bbuf, usem).wait()
    pltpu.sync_copy(bbuf, outb.at[pl.ds(base, BPW)])

    bufs = (wbuf0, wbuf1, wbuf2)
    gsems = (gs0, gs1, gs2)
    psems = (ps0, ps1, ps2)

    def g_desc(d, b):
        o = pl.multiple_of((d // NSL) * EG, EG)
        c = pl.multiple_of((d % NSL) * SR, SR)
        return pltpu.make_async_copy(
            wtab.at[widx_v.at[pl.ds(o, EG)], pl.ds(c, SR)], bufs[b], gsems[b])

    def p_desc(d, b):
        o = pl.multiple_of((d // NSL) * EG, EG)
        c = pl.multiple_of((d % NSL) * SR, SR)
        return pltpu.make_async_copy(
            bufs[b], outw.at[pl.ds(base + o, EG), pl.ds(c, SR)], psems[b])

    g_desc(0, 0).start()
    g_desc(1, 1).start()

    def step(i, carry):
        for k in range(3):
            d = i * 3 + k
            g_desc(d, k).wait()
            p_desc(d, k).start()
            dn = d + 2
            bn = (k + 2) % 3

            @pl.when(dn < NDMA)
            def _():
                @pl.when(d >= 1)
                def _():
                    p_desc(d - 1, bn).wait()

                g_desc(dn, bn).start()
        return carry

    lax.fori_loop(0, (NDMA - 1) // 3, step, 0)
    g_desc(NDMA - 1, 0).wait()
    p_desc(NDMA - 1, 0).start()
    p_desc(NDMA - 3, 1).wait()
    p_desc(NDMA - 2, 2).wait()
    p_desc(NDMA - 1, 0).wait()


def kernel(layer_index, selector, W0, b0, layers_distribution):
    ld = lax.dynamic_slice_in_dim(layers_distribution, layer_index, 1)
    selflat = selector.reshape(B * GROUPS)
    outw, outb = _sc_lookup(ld, selflat, W0, b0)
    return (outw, outb)


# speedup vs baseline: 3.8568x; 1.0057x over previous
import functools

import jax
import jax.numpy as jnp
from jax import lax
from jax.experimental import pallas as pl
from jax.experimental.pallas import tpu as pltpu
from jax.experimental.pallas import tpu_sc as plsc

GROUPS = 4
V = 1024
B = 1024
D = 256
EG = 8
SR = 16
NSL = D // SR
NC = 2
NS = 16
L = 16
NW = NC * NS
BPW = B // NW
NDMA = (BPW // EG) * NSL

_mesh = plsc.VectorSubcoreMesh(core_axis_name="c", subcore_axis_name="s")


@functools.partial(
    pl.kernel,
    mesh=_mesh,
    out_type=(
        jax.ShapeDtypeStruct((B, D, D), jnp.float32),
        jax.ShapeDtypeStruct((B, D), jnp.float32),
    ),
    scratch_types=[
        pltpu.VMEM((L,), jnp.int32),
        pltpu.VMEM((L,), jnp.int32),
        pltpu.VMEM((BPW,), jnp.int32),
        pltpu.VMEM((BPW,), jnp.int32),
        pltpu.VMEM((BPW, D), jnp.float32),
        pltpu.VMEM((EG, SR, D), jnp.float32),
        pltpu.VMEM((EG, SR, D), jnp.float32),
        pltpu.VMEM((EG, SR, D), jnp.float32),
        pltpu.SemaphoreType.DMA,
        pltpu.SemaphoreType.DMA,
        pltpu.SemaphoreType.DMA,
        pltpu.SemaphoreType.DMA,
        pltpu.SemaphoreType.DMA,
        pltpu.SemaphoreType.DMA,
        pltpu.SemaphoreType.DMA,
    ],
)
def _sc_lookup(ld, selflat, wtab, btab, outw, outb,
               z_v, g_v, sidx_v, widx_v, bbuf, wbuf0, wbuf1, wbuf2,
               usem, gs0, gs1, gs2, ps0, ps1, ps2):
    wid = lax.axis_index("s") * NC + lax.axis_index("c")
    base = pl.multiple_of(wid * BPW, BPW)
    iota = lax.iota(jnp.int32, L)

    z_v[...] = iota * 0
    pltpu.async_copy(ld.at[z_v], g_v, usem).wait()
    g = g_v[...]

    for h in range(BPW // L):
        sidx_v[pl.ds(h * L, L)] = (base + h * L + iota) * GROUPS + g
    pltpu.async_copy(selflat.at[sidx_v], widx_v, usem).wait()

    b_gather = pltpu.make_async_copy(btab.at[widx_v], bbuf, usem)
    b_gather.start()

    bufs = (wbuf0, wbuf1, wbuf2)
    gsems = (gs0, gs1, gs2)
    psems = (ps0, ps1, ps2)

    def g_desc(d, b):
        o = pl.multiple_of((d // NSL) * EG, EG)
        c = pl.multiple_of((d % NSL) * SR, SR)
        return pltpu.make_async_copy(
            wtab.at[widx_v.at[pl.ds(o, EG)], pl.ds(c, SR)], bufs[b], gsems[b])

    def p_desc(d, b):
        o = pl.multiple_of((d // NSL) * EG, EG)
        c = pl.multiple_of((d % NSL) * SR, SR)
        return pltpu.make_async_copy(
            bufs[b], outw.at[pl.ds(base + o, EG), pl.ds(c, SR)], psems[b])

    g_desc(0, 0).start()
    g_desc(1, 1).start()

    def step(i, carry):
        for k in range(3):
            d = i * 3 + k
            g_desc(d, k).wait()
            p_desc(d, k).start()
            dn = d + 2
            bn = (k + 2) % 3

            @pl.when(dn < NDMA)
            def _():
                @pl.when(d >= 1)
                def _():
                    p_desc(d - 1, bn).wait()

                g_desc(dn, bn).start()
        return carry

    lax.fori_loop(0, (NDMA - 1) // 3, step, 0)
    g_desc(NDMA - 1, 0).wait()
    p_desc(NDMA - 1, 0).start()
    b_gather.wait()
    pltpu.sync_copy(bbuf, outb.at[pl.ds(base, BPW)])
    p_desc(NDMA - 3, 1).wait()
    p_desc(NDMA - 2, 2).wait()
    p_desc(NDMA - 1, 0).wait()


def kernel(layer_index, selector, W0, b0, layers_distribution):
    ld = lax.dynamic_slice_in_dim(layers_distribution, layer_index, 1)
    selflat = selector.reshape(B * GROUPS)
    outw, outb = _sc_lookup(ld, selflat, W0, b0)
    return (outw, outb)
